# Initial kernel scaffold; baseline (speedup 1.0000x reference)
import jax
import jax.numpy as jnp
from jax.experimental import pallas as pl


def kernel(x, edge_attr, lg_x, lg_edge_attr, edge_index, lg_edge_index, batch, W_node, b_node, W_edge, b_edge, W_lg, b_lg, Wl, bl, Wg, bg, Wa, ba, gnw, gnb, gew, geb, W_attn, b_attn, Wh1, bh1, Wh2, bh2):
    raise NotImplementedError("write your pallas kernel here")


# speedup vs baseline: 1.6517x; 1.6517x over previous
import functools
import jax
import jax.numpy as jnp
from jax import lax
from jax.experimental import pallas as pl
from jax.experimental.pallas import tpu as pltpu
from jax.experimental.pallas import tpu_sc as plsc

N = 10000
E = 160000
ELG = 320000
H = 128
G = 256
L = 3

_NC = 2
_NS = 16


def _silu(v):
    return v * jax.nn.sigmoid(v)


@functools.lru_cache(maxsize=None)
def _sc_gather(T, B, C=400):
    nw = _NC * _NS
    bt = B // nw
    nit = bt // C
    mesh = plsc.VectorSubcoreMesh(core_axis_name="c", subcore_axis_name="s")

    @functools.partial(
        pl.kernel,
        mesh=mesh,
        out_type=jax.ShapeDtypeStruct((B, H), jnp.float32),
        scratch_types=[
            pltpu.VMEM((C,), jnp.int32),
            pltpu.VMEM((C, H), jnp.float32),
            pltpu.SemaphoreType.DMA,
        ],
    )
    def k(table_hbm, idx_hbm, out_hbm, idx_v, rows_v, sem):
        wid = lax.axis_index("s") * _NC + lax.axis_index("c")
        base = wid * bt

        def body(i, carry):
            off = base + i * C
            pltpu.sync_copy(idx_hbm.at[pl.ds(off, C)], idx_v)
            pltpu.async_copy(table_hbm.at[idx_v], rows_v, sem).wait()
            pltpu.sync_copy(rows_v, out_hbm.at[pl.ds(off, C)])
            return carry

        lax.fori_loop(0, nit, body, 0)

    return k


@functools.lru_cache(maxsize=None)
def _sc_scatter_add(M, K, D, KC, C):
    nw = _NC * _NS
    mt = M // nw
    nit = mt // C
    nchunks = -(-K // KC)
    R = KC + 128
    ZC = 64
    zr = R // 16
    dr = KC // 16
    DC = max(d for d in range(8, C + 1, 8) if dr % d == 0)
    assert R % (16 * ZC) == 0 and dr % 8 == 0
    mesh = plsc.VectorSubcoreMesh(core_axis_name="c", subcore_axis_name="s")

    scratch = [
        pltpu.VMEM_SHARED((R, D), jnp.float32),
        pltpu.VMEM((C, D), jnp.float32),
        pltpu.VMEM((C,), jnp.int32),
        pltpu.VMEM((1, C), jnp.int32),
        pltpu.VMEM((ZC, D), jnp.float32),
    ]

    @functools.partial(
        pl.kernel,
        mesh=mesh,
        out_type=jax.ShapeDtypeStruct((_NC * nchunks * KC, D), jnp.float32),
        scratch_types=scratch,
    )
    def k(data_hbm, idx_hbm, zeros_hbm, out_hbm, shared, data_v, idx_v, idxl_v,
          zbuf_v):
        cid = lax.axis_index("c")
        sid = lax.axis_index("s")
        base = cid * (M // _NC) + sid * mt
        pltpu.sync_copy(zeros_hbm, zbuf_v)

        def chunk(ch, carry0):
            cbase = ch * KC

            def zero(j, carry):
                pltpu.sync_copy(zbuf_v, shared.at[pl.ds(sid * zr + j * ZC, ZC)])
                return carry

            lax.fori_loop(0, zr // ZC, zero, 0)
            plsc.subcore_barrier()

            def body(i, carry):
                off = base + i * C
                pltpu.sync_copy(idx_hbm.at[pl.ds(off, C)], idx_v)
                pltpu.sync_copy(data_hbm.at[pl.ds(off, C)], data_v)
                def remap(j, c2):
                    v = idx_v[pl.ds(j * 16, 16)]
                    rel = v - cbase
                    oob = (rel < 0) | (rel >= KC)
                    idxl_v[0, pl.ds(j * 16, 16)] = jnp.where(oob, KC, rel)
                    return c2

                lax.fori_loop(0, C // 16, remap, 0)
                pltpu.sync_copy(data_v, shared.at[idxl_v.at[0]], add=True)
                return carry

            lax.fori_loop(0, nit, body, 0)
            plsc.subcore_barrier()

            def drain(j, carry):
                row = sid * dr + j * DC
                pltpu.sync_copy(shared.at[pl.ds(row, DC)], data_v.at[pl.ds(0, DC)])
                pltpu.sync_copy(
                    data_v.at[pl.ds(0, DC)],
                    out_hbm.at[pl.ds(cid * (nchunks * KC) + cbase + row, DC)],
                )
                return carry

            lax.fori_loop(0, dr // DC, drain, 0)
            plsc.subcore_barrier()
            return carry0

        lax.fori_loop(0, nchunks, chunk, 0)

    return k


def _scatter_add(data, idx, K, KC, C):
    s = jax.ops.segment_sum(data, idx, num_segments=K)
    return jnp.stack([s, jnp.zeros_like(s)])


def _embed_call(a, W, b, BR):
    M, Din = a.shape
    grid = M // BR

    def kern(a_ref, w_ref, b_ref, o_ref):
        o_ref[...] = _silu(
            jnp.dot(a_ref[...], w_ref[...], preferred_element_type=jnp.float32)
            + b_ref[...]
        )

    return pl.pallas_call(
        kern,
        grid=(grid,),
        in_specs=[
            pl.BlockSpec((BR, Din), lambda i: (i, 0)),
            pl.BlockSpec((Din, H), lambda i: (0, 0)),
            pl.BlockSpec((1, H), lambda i: (0, 0)),
        ],
        out_specs=pl.BlockSpec((BR, H), lambda i: (i, 0)),
        out_shape=jax.ShapeDtypeStruct((M, H), jnp.float32),
    )(a, W, b.reshape(1, H))


def _msg_call(lxs, lxd, lg_ea, WlA, WlB, wlc, bl, BR=2000):
    grid = ELG // BR

    def kern(s_ref, d_ref, ea_ref, wa_ref, wb_ref, wc_ref, b_ref, o_ref):
        acc = jnp.dot(s_ref[...], wa_ref[...], preferred_element_type=jnp.float32)
        acc += jnp.dot(d_ref[...], wb_ref[...], preferred_element_type=jnp.float32)
        acc += ea_ref[...] * wc_ref[...] + b_ref[...]
        o_ref[...] = _silu(acc)

    return pl.pallas_call(
        kern,
        grid=(grid,),
        in_specs=[
            pl.BlockSpec((BR, H), lambda i: (i, 0)),
            pl.BlockSpec((BR, H), lambda i: (i, 0)),
            pl.BlockSpec((BR, 1), lambda i: (i, 0)),
            pl.BlockSpec((H, H), lambda i: (0, 0)),
            pl.BlockSpec((H, H), lambda i: (0, 0)),
            pl.BlockSpec((1, H), lambda i: (0, 0)),
            pl.BlockSpec((1, H), lambda i: (0, 0)),
        ],
        out_specs=pl.BlockSpec((BR, H), lambda i: (i, 0)),
        out_shape=jax.ShapeDtypeStruct((ELG, H), jnp.float32),
    )(lxs, lxd, lg_ea, WlA, WlB, wlc.reshape(1, H), bl.reshape(1, H))


def _ln_in_kernel(t, g, b):
    m = jnp.mean(t, axis=-1, keepdims=True)
    d = t - m
    s = jnp.mean(d * d, axis=-1, keepdims=True)
    return d * lax.rsqrt(s + 1e-5) * g + b


def _edge_call(lx, ssum2, cnt2, e, gew, geb, Wg, bg, BR=2000):
    grid = E // BR

    def kern(lx_ref, ss_ref, cn_ref, e_ref, gw_ref, gb_ref, wg_ref, bgr_ref,
             lxo_ref, eno_ref, geo_ref):
        ssum = ss_ref[0] + ss_ref[1]
        cnt = cn_ref[0, :, :1] + cn_ref[1, :, :1]
        lx_new = lx_ref[...] + ssum / jnp.maximum(cnt, 1.0)
        en = _ln_in_kernel(e_ref[...] + lx_new, gw_ref[...], gb_ref[...])
        gate = jax.nn.sigmoid(
            jnp.dot(en, wg_ref[...], preferred_element_type=jnp.float32)
            + bgr_ref[...]
        )
        lxo_ref[...] = lx_new
        eno_ref[...] = en
        geo_ref[...] = gate * en

    o = jax.ShapeDtypeStruct((E, H), jnp.float32)
    return pl.pallas_call(
        kern,
        grid=(grid,),
        in_specs=[
            pl.BlockSpec((BR, H), lambda i: (i, 0)),
            pl.BlockSpec((2, BR, H), lambda i: (0, i, 0)),
            pl.BlockSpec((2, BR, H), lambda i: (0, i, 0)),
            pl.BlockSpec((BR, H), lambda i: (i, 0)),
            pl.BlockSpec((1, H), lambda i: (0, 0)),
            pl.BlockSpec((1, H), lambda i: (0, 0)),
            pl.BlockSpec((H, H), lambda i: (0, 0)),
            pl.BlockSpec((1, H), lambda i: (0, 0)),
        ],
        out_specs=[pl.BlockSpec((BR, H), lambda i: (i, 0))] * 3,
        out_shape=[o, o, o],
    )(lx, ssum2, cnt2, e, gew.reshape(1, H), geb.reshape(1, H), Wg,
      bg.reshape(1, H))


def _node_call(x, agg2, WaT, WaB, ba, gnw, gnb, BR=400):
    grid = N // BR

    def kern(x_ref, ag_ref, wt_ref, wb_ref, ba_ref, gw_ref, gb_ref, o_ref):
        agg = ag_ref[0] + ag_ref[1]
        h = jnp.dot(x_ref[...], wt_ref[...], preferred_element_type=jnp.float32)
        h += jnp.dot(agg, wb_ref[...], preferred_element_type=jnp.float32)
        h = _silu(h + ba_ref[...])
        o_ref[...] = _ln_in_kernel(x_ref[...] + h, gw_ref[...], gb_ref[...])

    return pl.pallas_call(
        kern,
        grid=(grid,),
        in_specs=[
            pl.BlockSpec((BR, H), lambda i: (i, 0)),
            pl.BlockSpec((2, BR, H), lambda i: (0, i, 0)),
            pl.BlockSpec((H, H), lambda i: (0, 0)),
            pl.BlockSpec((H, H), lambda i: (0, 0)),
            pl.BlockSpec((1, H), lambda i: (0, 0)),
            pl.BlockSpec((1, H), lambda i: (0, 0)),
            pl.BlockSpec((1, H), lambda i: (0, 0)),
        ],
        out_specs=pl.BlockSpec((BR, H), lambda i: (i, 0)),
        out_shape=jax.ShapeDtypeStruct((N, H), jnp.float32),
    )(x, agg2, WaT, WaB, ba.reshape(1, H), gnw.reshape(1, H), gnb.reshape(1, H))


def _raw_call(x, W_attn, b_attn, BR=400):
    grid = N // BR
    Wbc = jnp.broadcast_to(W_attn, (H, H))

    def kern(x_ref, w_ref, b_ref, o_ref):
        o_ref[...] = (
            jnp.dot(x_ref[...], w_ref[...], preferred_element_type=jnp.float32)
            + b_ref[0, 0]
        )

    return pl.pallas_call(
        kern,
        grid=(grid,),
        in_specs=[
            pl.BlockSpec((BR, H), lambda i: (i, 0)),
            pl.BlockSpec((H, H), lambda i: (0, 0)),
            pl.BlockSpec((1, 1), lambda i: (0, 0)),
        ],
        out_specs=pl.BlockSpec((BR, H), lambda i: (i, 0)),
        out_shape=jax.ShapeDtypeStruct((N, H), jnp.float32),
    )(x, Wbc, b_attn.reshape(1, 1))


def _smax_call(raw, batch_c3, BR=400):
    grid = N // BR

    def kern(r_ref, b_ref, o_ref):
        i = pl.program_id(0)
        bc = b_ref[0]
        gi = lax.broadcasted_iota(jnp.int32, (BR, G), 1)
        onehot = bc == gi
        masked = jnp.where(onehot, r_ref[:, :1], -jnp.inf)
        m = jnp.max(masked, axis=0, keepdims=True)

        @pl.when(i == 0)
        def _():
            o_ref[...] = jnp.full((1, G), -jnp.inf, jnp.float32)

        o_ref[...] = jnp.maximum(o_ref[...], m)

    return pl.pallas_call(
        kern,
        grid=(grid,),
        in_specs=[
            pl.BlockSpec((BR, H), lambda i: (i, 0)),
            pl.BlockSpec((1, BR, 1), lambda i: (i, 0, 0)),
        ],
        out_specs=pl.BlockSpec((1, G), lambda i: (0, 0)),
        out_shape=jax.ShapeDtypeStruct((1, G), jnp.float32),
    )(raw, batch_c3)


def _denom_call(raw, smax, batch_c3, BR=400):
    grid = N // BR

    def kern(r_ref, s_ref, b_ref, o_ref):
        i = pl.program_id(0)
        bc = b_ref[0]
        gi = lax.broadcasted_iota(jnp.int32, (BR, G), 1)
        onehot = bc == gi
        sm = s_ref[...]
        sm = jnp.where(jnp.isfinite(sm), sm, 0.0)
        srow = jnp.max(jnp.where(onehot, sm, -jnp.inf), axis=1, keepdims=True)
        ex = jnp.exp(r_ref[:, :1] - srow)
        part = jnp.sum(jnp.where(onehot, ex, 0.0), axis=0, keepdims=True)

        @pl.when(i == 0)
        def _():
            o_ref[...] = jnp.zeros((1, G), jnp.float32)

        o_ref[...] += part

    return pl.pallas_call(
        kern,
        grid=(grid,),
        in_specs=[
            pl.BlockSpec((BR, H), lambda i: (i, 0)),
            pl.BlockSpec((1, G), lambda i: (0, 0)),
            pl.BlockSpec((1, BR, 1), lambda i: (i, 0, 0)),
        ],
        out_specs=pl.BlockSpec((1, G), lambda i: (0, 0)),
        out_shape=jax.ShapeDtypeStruct((1, G), jnp.float32),
    )(raw, smax, batch_c3)


def _gr_call(raw, smax, denom, batch_c3, batch_r3, x, BR=400):
    grid = N // BR

    def kern(r_ref, s_ref, d_ref, bc_ref, br_ref, x_ref, o_ref):
        i = pl.program_id(0)
        bc = bc_ref[0]
        gi = lax.broadcasted_iota(jnp.int32, (BR, G), 1)
        onehot = bc == gi
        sm = s_ref[...]
        sm = jnp.where(jnp.isfinite(sm), sm, 0.0)
        srow = jnp.max(jnp.where(onehot, sm, -jnp.inf), axis=1, keepdims=True)
        drow = jnp.max(jnp.where(onehot, d_ref[...], -jnp.inf), axis=1,
                       keepdims=True)
        ex = jnp.exp(r_ref[:, :1] - srow)
        score = ex / (drow + 1e-16)
        br = br_ref[0]
        gi2 = lax.broadcasted_iota(jnp.int32, (G, BR), 0)
        onehot_t = (br == gi2).astype(jnp.float32)
        part = jnp.dot(onehot_t, score * x_ref[...],
                       preferred_element_type=jnp.float32)

        @pl.when(i == 0)
        def _():
            o_ref[...] = jnp.zeros((G, H), jnp.float32)

        o_ref[...] += part

    return pl.pallas_call(
        kern,
        grid=(grid,),
        in_specs=[
            pl.BlockSpec((BR, H), lambda i: (i, 0)),
            pl.BlockSpec((1, G), lambda i: (0, 0)),
            pl.BlockSpec((1, G), lambda i: (0, 0)),
            pl.BlockSpec((1, BR, 1), lambda i: (i, 0, 0)),
            pl.BlockSpec((1, 1, BR), lambda i: (i, 0, 0)),
            pl.BlockSpec((BR, H), lambda i: (i, 0)),
        ],
        out_specs=pl.BlockSpec((G, H), lambda i: (0, 0)),
        out_shape=jax.ShapeDtypeStruct((G, H), jnp.float32),
    )(raw, smax, denom, batch_c3, batch_r3, x)


def _head_call(gr, Wh1, bh1, Wh2, bh2):
    Wh1p = jnp.zeros((H, H), jnp.float32).at[:, : H // 2].set(Wh1)
    bh1p = jnp.zeros((1, H), jnp.float32).at[0, : H // 2].set(bh1)
    Wh2p = jnp.zeros((H, H), jnp.float32).at[: H // 2, :1].set(Wh2)

    def kern(g_ref, w1_ref, b1_ref, w2_ref, b2_ref, o_ref):
        h = _silu(
            jnp.dot(g_ref[...], w1_ref[...], preferred_element_type=jnp.float32)
            + b1_ref[...]
        )
        o_ref[...] = (
            jnp.dot(h, w2_ref[...], preferred_element_type=jnp.float32)
            + b2_ref[0, 0]
        )

    return pl.pallas_call(
        kern,
        in_specs=[
            pl.BlockSpec((G, H), lambda: (0, 0)),
            pl.BlockSpec((H, H), lambda: (0, 0)),
            pl.BlockSpec((1, H), lambda: (0, 0)),
            pl.BlockSpec((H, H), lambda: (0, 0)),
            pl.BlockSpec((1, 1), lambda: (0, 0)),
        ],
        out_specs=pl.BlockSpec((G, H), lambda: (0, 0)),
        out_shape=jax.ShapeDtypeStruct((G, H), jnp.float32),
    )(gr, Wh1p, bh1p, Wh2p, bh2.reshape(1, 1))


def kernel(x, edge_attr, lg_x, lg_edge_attr, edge_index, lg_edge_index, batch,
           W_node, b_node, W_edge, b_edge, W_lg, b_lg, Wl, bl, Wg, bg, Wa, ba,
           gnw, gnb, gew, geb, W_attn, b_attn, Wh1, bh1, Wh2, bh2):
    src = lg_edge_index[0]
    dst = lg_edge_index[1]
    col = edge_index[1]

    x0 = _embed_call(x, W_node, b_node, 400)
    e = _embed_call(edge_attr, W_edge, b_edge, 2000)
    lx = _embed_call(lg_x, W_lg, b_lg, 2000)

    ones = jnp.ones((ELG, H), jnp.float32)
    cnt2 = _scatter_add(ones, dst, E, KC=8064, C=80)

    gidx = jnp.concatenate([src, dst])
    for i in range(L):
        g = _sc_gather(E, 2 * ELG, C=80)(lx, gidx)
        lxs, lxd = g[:ELG], g[ELG:]
        msg = _msg_call(lxs, lxd, lg_edge_attr, Wl[i, :H], Wl[i, H:2 * H],
                        Wl[i, 2 * H], bl[i])
        ssum2 = _scatter_add(msg, dst, E, KC=8064, C=80)
        lx, en, ge = _edge_call(lx, ssum2, cnt2, e, gew[i], geb[i], Wg[i], bg[i])
        agg2 = _scatter_add(ge, col, N, KC=1920, C=40)
        x0 = _node_call(x0, agg2, Wa[i, :H], Wa[i, H:], ba[i], gnw[i], gnb[i])
        e = en

    raw = _raw_call(x0, W_attn, b_attn)
    batch_c3 = batch.reshape(N // 400, 400, 1)
    batch_r3 = batch.reshape(N // 400, 1, 400)
    smax = _smax_call(raw, batch_c3)
    denom = _denom_call(raw, smax, batch_c3)
    gr = _gr_call(raw, smax, denom, batch_c3, batch_r3, x0)
    out = _head_call(gr, Wh1, bh1, Wh2, bh2)
    return out[:, :1]
